# Initial kernel scaffold; baseline (speedup 1.0000x reference)
import jax
import jax.numpy as jnp
from jax.experimental import pallas as pl


def kernel(reads, edge_index, overlap_similarity, overlap_length, Wc, bc, We, be, WA, WB, WC, bE, WU, WV, bH, Wd1, bd1, Wd2, bd2):
    raise NotImplementedError("write your pallas kernel here")


# speedup vs baseline: 1.5200x; 1.5200x over previous
import jax
import jax.numpy as jnp
from jax import lax
from jax.experimental import pallas as pl
from jax.experimental.pallas import tpu as pltpu
from jax.experimental.pallas import tpu_sc as plsc

N = 10000
E = 320000
D = 128
K = 16
READ_LEN = 128
NUM_LAYERS = 4
H = D // 2
T = (READ_LEN - K) // 4 + 1

NSUB = 16
NCORE = 2
EH = E // 2
EB = 32
EBD = 80
NBLK_L = EH // EB
NBLK_ALL = EH // EBD
NZ = 624

F32 = jnp.float32
HIGH = lax.Precision.HIGHEST


def _proj(h, wav_ref, wb_ref, wu_ref, bh_ref):
  tav0 = jnp.dot(h, wav_ref[0], preferred_element_type=F32)
  tav1 = jnp.dot(h, wav_ref[1], preferred_element_type=F32)
  hb = jnp.dot(h, wb_ref[...], preferred_element_type=F32)
  hbs = jnp.concatenate([hb[:, H:], hb[:, :H]], axis=1)
  hu = jnp.dot(h, wu_ref[...], preferred_element_type=F32)
  hu = hu + bh_ref[...]
  return tav0, tav1, hb, hbs, hu


def _encoder_body(reads_ref, wck_ref, bc_ref, wav_ref, wb_ref, wu_ref, bh_ref,
                  h_ref, tav_ref, tb_ref, hu_ref):
  bn = reads_ref.shape[0]
  acc = jnp.zeros((bn, D), F32)
  bc = bc_ref[...]
  w2 = jnp.concatenate([wck_ref[c] for c in range(4)], axis=0)
  for t in range(T):
    win = jnp.concatenate([reads_ref[:, c, 4 * t:4 * t + K] for c in range(4)],
                          axis=1)
    s = jnp.dot(win, w2, preferred_element_type=F32)
    acc = acc + jnp.maximum(s + bc, 0.0)
  h = acc * (1.0 / T)
  tav0, tav1, hb, hbs, hu = _proj(h, wav_ref, wb_ref, wu_ref, bh_ref)
  h_ref[...] = h
  tav_ref[0] = tav0
  tav_ref[1] = tav1
  tb_ref[0] = hb
  tb_ref[1] = hbs
  hu_ref[...] = hu


def _node_body(h_ref, hu_ref, ag_ref, wav_ref, wb_ref, wu_ref, bh_ref,
               h2_ref, tav_ref, tb_ref, hu2_ref):
  agg = jnp.concatenate([ag_ref[0, :, :H], ag_ref[1, :, :H]], axis=1)
  nrm = jnp.concatenate([ag_ref[0, :, H:], ag_ref[1, :, H:]], axis=1)
  h = h_ref[...] + jnp.maximum(hu_ref[...] + agg / (nrm + 1e-6), 0.0)
  tav0, tav1, hb, hbs, hu = _proj(h, wav_ref, wb_ref, wu_ref, bh_ref)
  h2_ref[...] = h
  tav_ref[0] = tav0
  tav_ref[1] = tav1
  tb_ref[0] = hb
  tb_ref[1] = hbs
  hu2_ref[...] = hu


def _pair_split(lo, hi):
  p0 = jnp.concatenate([lo[:, :H], hi[:, :H]], axis=1)
  p1 = jnp.concatenate([lo[:, H:], hi[:, H:]], axis=1)
  return p0, p1


def _pair_join(p0, p1):
  lo = jnp.concatenate([p0[:, :H], p1[:, :H]], axis=1)
  hi = jnp.concatenate([p0[:, H:], p1[:, H:]], axis=1)
  return lo, hi


def _edge_mm_body(ep_ref, wc_ref, be_ref, ewp_ref):
  lo, hi = _pair_join(ep_ref[0], ep_ref[1])
  ew_lo = jnp.dot(lo, wc_ref[...], preferred_element_type=F32) + be_ref[...]
  ew_hi = jnp.dot(hi, wc_ref[...], preferred_element_type=F32) + be_ref[...]
  p0, p1 = _pair_split(ew_lo, ew_hi)
  ewp_ref[0] = p0
  ewp_ref[1] = p1


def _edge0_body(sim_lo, sim_hi, ln_lo, ln_hi, we_ref, be0_ref, wc_ref, be_ref,
                ep_ref, ewp_ref):
  def enc(sim, ln):
    return jnp.maximum(sim[...] * we_ref[0:1, :] + ln[...] * we_ref[1:2, :]
                       + be0_ref[...], 0.0)
  e_lo = enc(sim_lo, ln_lo)
  e_hi = enc(sim_hi, ln_hi)
  p0, p1 = _pair_split(e_lo, e_hi)
  ep_ref[0] = p0
  ep_ref[1] = p1
  ew_lo = jnp.dot(e_lo, wc_ref[...], preferred_element_type=F32) + be_ref[...]
  ew_hi = jnp.dot(e_hi, wc_ref[...], preferred_element_type=F32) + be_ref[...]
  q0, q1 = _pair_split(ew_lo, ew_hi)
  ewp_ref[0] = q0
  ewp_ref[1] = q1


def _dec_reduce_body(pacc_ref, bd2_ref, p_ref):
  p_ref[...] = jnp.sum(pacc_ref[...], axis=1, keepdims=True) + bd2_ref[...]


def _sc_layer_body(src, dst, tav, tbsw, ep, ewp, eop, aggsig,
                   slv, shv, dlv, dhv, ialo, iahi, iblo, ibhi,
                   avlo, avhi, blo, bhi, epb, ewpb,
                   shared, sem1, sem2, sem3, sem4):
  c = lax.axis_index("c")
  s = lax.axis_index("s")
  cn = c * N

  def zrow(r, _):
    for k2 in range(8):
      epb[r, pl.ds(k2 * 16, 16)] = jnp.zeros((16,), F32)
    return 0
  lax.fori_loop(0, EB, zrow, 0)
  nbase = s * NZ
  for z in range(0, NZ, EB):
    rows = min(EB, NZ - z)
    pltpu.sync_copy(epb.at[pl.ds(0, rows)], shared.at[pl.ds(nbase + z, rows)])

  @pl.when(s == NSUB - 1)
  def _():
    pltpu.sync_copy(epb.at[pl.ds(0, N - NZ * NSUB)],
                    shared.at[pl.ds(NZ * NSUB, N - NZ * NSUB)])
  plsc.subcore_barrier()

  def do_block(j):
    base = j * EB
    pltpu.sync_copy(src.at[pl.ds(base, EB)], slv)
    pltpu.sync_copy(src.at[pl.ds(EH + base, EB)], shv)
    pltpu.sync_copy(dst.at[pl.ds(base, EB)], dlv)
    pltpu.sync_copy(dst.at[pl.ds(EH + base, EB)], dhv)
    for k2 in range(EB // 16):
      sl = pl.ds(k2 * 16, 16)
      ialo[sl] = slv[sl] + cn
      iahi[sl] = shv[sl] + cn
      iblo[sl] = dlv[sl] + cn
      ibhi[sl] = dhv[sl] + cn
    g1 = pltpu.async_copy(tav.at[ialo], avlo, sem1)
    g2 = pltpu.async_copy(tav.at[iahi], avhi, sem2)
    g3 = pltpu.async_copy(tbsw.at[iblo], blo, sem3)
    g4 = pltpu.async_copy(tbsw.at[ibhi], bhi, sem4)
    pltpu.sync_copy(ep.at[c, pl.ds(base, EB)], epb)
    pltpu.sync_copy(ewp.at[c, pl.ds(base, EB)], ewpb)
    g1.wait()
    g2.wait()
    g3.wait()
    g4.wait()

    def pair(r, _):
      for k2 in range(H // 16):
        sl = pl.ds(k2 * 16, 16)
        sv = pl.ds(H + k2 * 16, 16)
        en = avlo[r, sl] + blo[r, sl] + ewpb[r, sl]
        sg = 1.0 / (1.0 + jnp.exp(-en))
        avlo[r, sl] = sg * avlo[r, sv]
        avlo[r, sv] = sg
        epb[r, sl] = epb[r, sl] + jnp.maximum(en, 0.0)
        en2 = avhi[r, sl] + bhi[r, sl] + ewpb[r, sv]
        sg2 = 1.0 / (1.0 + jnp.exp(-en2))
        avhi[r, sl] = sg2 * avhi[r, sv]
        avhi[r, sv] = sg2
        epb[r, sv] = epb[r, sv] + jnp.maximum(en2, 0.0)
      return 0
    lax.fori_loop(0, EB, pair, 0)

    pltpu.sync_copy(epb, eop.at[c, pl.ds(base, EB)])
    pltpu.sync_copy(avlo, shared.at[dlv], add=True)
    pltpu.sync_copy(avhi, shared.at[dhv], add=True)

  def block(i, _):
    do_block(s + i * NSUB)
    return 0
  nfull = NBLK_L // NSUB
  lax.fori_loop(0, nfull, block, 0)
  nrem = NBLK_L - nfull * NSUB

  @pl.when(s < nrem)
  def _():
    do_block(nfull * NSUB + s)
  plsc.subcore_barrier()
  pltpu.sync_copy(shared.at[pl.ds(nbase, NZ)], aggsig.at[c, pl.ds(nbase, NZ)])

  @pl.when(s == NSUB - 1)
  def _():
    pltpu.sync_copy(shared.at[pl.ds(NZ * NSUB, N - NZ * NSUB)],
                    aggsig.at[c, pl.ds(NZ * NSUB, N - NZ * NSUB)])


def _sc_dec_body(src, dst, da, db, qp, wd2, pacc,
                 slv, shv, dlv, dhv, alo, ahi, blo, bhi, q0b, q1b,
                 wv, pblo, pbhi, sem1, sem2, sem3, sem4):
  c = lax.axis_index("c")
  s = lax.axis_index("s")
  w = s * NCORE + c
  pltpu.sync_copy(wd2, wv)

  def do_block(jb):
    base = jb * EBD
    pltpu.sync_copy(src.at[pl.ds(base, EBD)], slv)
    pltpu.sync_copy(src.at[pl.ds(EH + base, EBD)], shv)
    pltpu.sync_copy(dst.at[pl.ds(base, EBD)], dlv)
    pltpu.sync_copy(dst.at[pl.ds(EH + base, EBD)], dhv)
    g1 = pltpu.async_copy(da.at[slv], alo, sem1)
    g2 = pltpu.async_copy(da.at[shv], ahi, sem2)
    g3 = pltpu.async_copy(db.at[dlv], blo, sem3)
    g4 = pltpu.async_copy(db.at[dhv], bhi, sem4)
    pltpu.sync_copy(qp.at[0, pl.ds(base, EBD)], q0b)
    pltpu.sync_copy(qp.at[1, pl.ds(base, EBD)], q1b)
    g1.wait()
    g2.wait()
    g3.wait()
    g4.wait()

    def pair(r, _):
      acc_lo = jnp.zeros((16,), F32)
      acc_hi = jnp.zeros((16,), F32)
      for k2 in range(D // 16):
        sl = pl.ds(k2 * 16, 16)
        if k2 < 4:
          q_lo = q0b[r, pl.ds(k2 * 16, 16)]
          q_hi = q0b[r, pl.ds(H + k2 * 16, 16)]
        else:
          q_lo = q1b[r, pl.ds((k2 - 4) * 16, 16)]
          q_hi = q1b[r, pl.ds(H + (k2 - 4) * 16, 16)]
        wk = wv[sl]
        acc_lo = acc_lo + wk * jnp.maximum(q_lo + alo[r, sl] + blo[r, sl], 0.0)
        acc_hi = acc_hi + wk * jnp.maximum(q_hi + ahi[r, sl] + bhi[r, sl], 0.0)
      pblo[r, :] = acc_lo
      pbhi[r, :] = acc_hi
      return 0
    lax.fori_loop(0, EBD, pair, 0)
    pltpu.sync_copy(pblo, pacc.at[pl.ds(base, EBD)])
    pltpu.sync_copy(pbhi, pacc.at[pl.ds(EH + base, EBD)])

  def block(i, _):
    do_block(w + i * (NCORE * NSUB))
    return 0
  nfull = NBLK_ALL // (NCORE * NSUB)
  lax.fori_loop(0, nfull, block, 0)
  nrem = NBLK_ALL - nfull * (NCORE * NSUB)

  @pl.when(w < nrem)
  def _():
    do_block(nfull * (NCORE * NSUB) + w)


def _mk_sc_layer():
  mesh = plsc.VectorSubcoreMesh(core_axis_name="c", subcore_axis_name="s")
  return pl.kernel(
      _sc_layer_body,
      out_type=[jax.ShapeDtypeStruct((2, EH, D), F32),
                jax.ShapeDtypeStruct((2, N, D), F32)],
      mesh=mesh,
      scratch_types=[
          pltpu.VMEM((EB,), jnp.int32),
          pltpu.VMEM((EB,), jnp.int32),
          pltpu.VMEM((EB,), jnp.int32),
          pltpu.VMEM((EB,), jnp.int32),
          pltpu.VMEM((EB,), jnp.int32),
          pltpu.VMEM((EB,), jnp.int32),
          pltpu.VMEM((EB,), jnp.int32),
          pltpu.VMEM((EB,), jnp.int32),
          pltpu.VMEM((EB, D), F32),
          pltpu.VMEM((EB, D), F32),
          pltpu.VMEM((EB, D), F32),
          pltpu.VMEM((EB, D), F32),
          pltpu.VMEM((EB, D), F32),
          pltpu.VMEM((EB, D), F32),
          pltpu.VMEM_SHARED((N, D), F32),
          pltpu.SemaphoreType.DMA,
          pltpu.SemaphoreType.DMA,
          pltpu.SemaphoreType.DMA,
          pltpu.SemaphoreType.DMA,
      ],
  )


def _mk_sc_dec():
  mesh = plsc.VectorSubcoreMesh(core_axis_name="c", subcore_axis_name="s")
  return pl.kernel(
      _sc_dec_body,
      out_type=jax.ShapeDtypeStruct((E, 16), F32),
      mesh=mesh,
      scratch_types=[
          pltpu.VMEM((EBD,), jnp.int32),
          pltpu.VMEM((EBD,), jnp.int32),
          pltpu.VMEM((EBD,), jnp.int32),
          pltpu.VMEM((EBD,), jnp.int32),
          pltpu.VMEM((EBD, D), F32),
          pltpu.VMEM((EBD, D), F32),
          pltpu.VMEM((EBD, D), F32),
          pltpu.VMEM((EBD, D), F32),
          pltpu.VMEM((EBD, D), F32),
          pltpu.VMEM((EBD, D), F32),
          pltpu.VMEM((D,), F32),
          pltpu.VMEM((EBD, 16), F32),
          pltpu.VMEM((EBD, 16), F32),
          pltpu.SemaphoreType.DMA,
          pltpu.SemaphoreType.DMA,
          pltpu.SemaphoreType.DMA,
          pltpu.SemaphoreType.DMA,
      ],
  )


def kernel(reads, edge_index, overlap_similarity, overlap_length, Wc, bc, We,
           be, WA, WB, WC, bE, WU, WV, bH, Wd1, bd1, Wd2, bd2):
  BN = 1000
  BEP = 400
  grid_n = N // BN
  grid_ep = EH // BEP

  wck = jnp.transpose(Wc, (1, 2, 0))
  def mk_wav(l):
    return jnp.stack([
        jnp.concatenate([WA[l][:, :H], WV[l][:, :H]], axis=1),
        jnp.concatenate([WA[l][:, H:], WV[l][:, H:]], axis=1)])
  wav_dec = jnp.stack([Wd1[:D], Wd1[D:2 * D]])

  src = edge_index[0]
  dst = edge_index[1]
  sim = overlap_similarity.reshape(E, 1)
  ln = overlap_length.reshape(E, 1)

  row = lambda *shape: pl.BlockSpec(
      shape, lambda i: tuple([i] + [0] * (len(shape) - 1)))
  row2 = lambda *shape: pl.BlockSpec(
      shape, lambda i: tuple([0, i] + [0] * (len(shape) - 2)))
  whole = lambda *shape: pl.BlockSpec(shape, lambda i: (0,) * len(shape))
  hi_blocks = grid_ep

  enc = pl.pallas_call(
      _encoder_body,
      grid=(grid_n,),
      in_specs=[row(BN, 4, READ_LEN), whole(4, K, D), whole(D),
                whole(2, D, D), whole(D, D), whole(D, D), whole(D)],
      out_specs=[row(BN, D), row2(2, BN, D), row2(2, BN, D), row(BN, D)],
      out_shape=[jax.ShapeDtypeStruct((N, D), F32),
                 jax.ShapeDtypeStruct((2, N, D), F32),
                 jax.ShapeDtypeStruct((2, N, D), F32),
                 jax.ShapeDtypeStruct((N, D), F32)],
  )
  h, tav, tbsw, hu = enc(reads, wck, bc, mk_wav(0), WB[0], WU[0], bH[0])

  lo_spec = pl.BlockSpec((BEP, 1), lambda i: (i, 0))
  hi_spec = pl.BlockSpec((BEP, 1), lambda i: (i + hi_blocks, 0))
  edge0 = pl.pallas_call(
      _edge0_body,
      grid=(grid_ep,),
      in_specs=[lo_spec, hi_spec, lo_spec, hi_spec, whole(2, D), whole(D),
                whole(D, D), whole(D)],
      out_specs=[row2(2, BEP, D), row2(2, BEP, D)],
      out_shape=[jax.ShapeDtypeStruct((2, EH, D), F32),
                 jax.ShapeDtypeStruct((2, EH, D), F32)],
  )
  ep, ewp = edge0(sim, sim, ln, ln, We, be, WC[0], bE[0])

  edge_mm = pl.pallas_call(
      _edge_mm_body,
      grid=(grid_ep,),
      in_specs=[row2(2, BEP, D), whole(D, D), whole(D)],
      out_specs=row2(2, BEP, D),
      out_shape=jax.ShapeDtypeStruct((2, EH, D), F32),
  )

  node = pl.pallas_call(
      _node_body,
      grid=(grid_n,),
      in_specs=[row(BN, D), row(BN, D), row2(2, BN, D),
                whole(2, D, D), whole(D, D), whole(D, D), whole(D)],
      out_specs=[row(BN, D), row2(2, BN, D), row2(2, BN, D), row(BN, D)],
      out_shape=[jax.ShapeDtypeStruct((N, D), F32),
                 jax.ShapeDtypeStruct((2, N, D), F32),
                 jax.ShapeDtypeStruct((2, N, D), F32),
                 jax.ShapeDtypeStruct((N, D), F32)],
  )

  sc_layer = _mk_sc_layer()
  for l in range(NUM_LAYERS):
    tav2 = tav.reshape(2 * N, D)
    tbsw2 = tbsw.reshape(2 * N, D)
    ep, aggsig = sc_layer(src, dst, tav2, tbsw2, ep, ewp)
    if l < NUM_LAYERS - 1:
      h, tav, tbsw, hu = node(h, hu, aggsig, mk_wav(l + 1), WB[l + 1],
                              WU[l + 1], bH[l + 1])
      ewp = edge_mm(ep, WC[l + 1], bE[l + 1])
    else:
      zD = jnp.zeros((D, D), F32)
      _, dadb, _, _ = node(h, hu, aggsig, wav_dec, zD, zD, jnp.zeros((D,), F32))
      qp = edge_mm(ep, Wd1[2 * D:], bd1)

  sc_dec = _mk_sc_dec()
  wd2f = Wd2.reshape(D)
  pacc = sc_dec(src, dst, dadb[0], dadb[1], qp, wd2f)

  dec_reduce = pl.pallas_call(
      _dec_reduce_body,
      grid=(E // 1000,),
      in_specs=[row(1000, 16), whole(1)],
      out_specs=row(1000, 1),
      out_shape=jax.ShapeDtypeStruct((E, 1), F32),
  )
  return dec_reduce(pacc, bd2)


# speedup vs baseline: 2.3598x; 1.5525x over previous
import jax
import jax.numpy as jnp
from jax import lax
from jax.experimental import pallas as pl
from jax.experimental.pallas import tpu as pltpu
from jax.experimental.pallas import tpu_sc as plsc

N = 10000
E = 320000
D = 128
K = 16
READ_LEN = 128
NUM_LAYERS = 4
H = D // 2
T = (READ_LEN - K) // 4 + 1

NSUB = 16
NCORE = 2
EH = E // 2
EB = 64
EBD = 80
NBLK_L = EH // EB
NBLK_ALL = EH // EBD
NZ = 624

F32 = jnp.float32
HIGH = lax.Precision.HIGHEST


def _proj(h, wav_ref, wb_ref, wu_ref, bh_ref):
  tav0 = jnp.dot(h, wav_ref[0], preferred_element_type=F32)
  tav1 = jnp.dot(h, wav_ref[1], preferred_element_type=F32)
  hb = jnp.dot(h, wb_ref[...], preferred_element_type=F32)
  hbs = jnp.concatenate([hb[:, H:], hb[:, :H]], axis=1)
  hu = jnp.dot(h, wu_ref[...], preferred_element_type=F32)
  hu = hu + bh_ref[...]
  return tav0, tav1, hb, hbs, hu


def _encoder_body(reads_ref, wck_ref, bc_ref, wav_ref, wb_ref, wu_ref, bh_ref,
                  h_ref, tav_ref, tb_ref, hu_ref):
  bn = reads_ref.shape[0]
  acc = jnp.zeros((bn, D), F32)
  bc = bc_ref[...]
  w2 = jnp.concatenate([wck_ref[c] for c in range(4)], axis=0)
  for t in range(T):
    win = jnp.concatenate([reads_ref[:, c, 4 * t:4 * t + K] for c in range(4)],
                          axis=1)
    s = jnp.dot(win, w2, preferred_element_type=F32)
    acc = acc + jnp.maximum(s + bc, 0.0)
  h = acc * (1.0 / T)
  tav0, tav1, hb, hbs, hu = _proj(h, wav_ref, wb_ref, wu_ref, bh_ref)
  h_ref[...] = h
  tav_ref[0] = tav0
  tav_ref[1] = tav1
  tb_ref[0] = hb
  tb_ref[1] = hbs
  hu_ref[...] = hu


def _node_body(h_ref, hu_ref, ag_ref, wav_ref, wb_ref, wu_ref, bh_ref,
               h2_ref, tav_ref, tb_ref, hu2_ref):
  agg = jnp.concatenate([ag_ref[0, :, :H], ag_ref[1, :, :H]], axis=1)
  nrm = jnp.concatenate([ag_ref[0, :, H:], ag_ref[1, :, H:]], axis=1)
  h = h_ref[...] + jnp.maximum(hu_ref[...] + agg / (nrm + 1e-6), 0.0)
  tav0, tav1, hb, hbs, hu = _proj(h, wav_ref, wb_ref, wu_ref, bh_ref)
  h2_ref[...] = h
  tav_ref[0] = tav0
  tav_ref[1] = tav1
  tb_ref[0] = hb
  tb_ref[1] = hbs
  hu2_ref[...] = hu


def _pair_split(lo, hi):
  p0 = jnp.concatenate([lo[:, :H], hi[:, :H]], axis=1)
  p1 = jnp.concatenate([lo[:, H:], hi[:, H:]], axis=1)
  return p0, p1


def _pair_join(p0, p1):
  lo = jnp.concatenate([p0[:, :H], p1[:, :H]], axis=1)
  hi = jnp.concatenate([p0[:, H:], p1[:, H:]], axis=1)
  return lo, hi


def _edge_mm_body(ep_ref, wc_ref, be_ref, ewp_ref):
  lo, hi = _pair_join(ep_ref[0], ep_ref[1])
  ew_lo = jnp.dot(lo, wc_ref[...], preferred_element_type=F32) + be_ref[...]
  ew_hi = jnp.dot(hi, wc_ref[...], preferred_element_type=F32) + be_ref[...]
  p0, p1 = _pair_split(ew_lo, ew_hi)
  ewp_ref[0] = p0
  ewp_ref[1] = p1


def _edge0_body(sim_lo, sim_hi, ln_lo, ln_hi, we_ref, be0_ref, wc_ref, be_ref,
                ep_ref, ewp_ref):
  def enc(sim, ln):
    return jnp.maximum(sim[...] * we_ref[0:1, :] + ln[...] * we_ref[1:2, :]
                       + be0_ref[...], 0.0)
  e_lo = enc(sim_lo, ln_lo)
  e_hi = enc(sim_hi, ln_hi)
  p0, p1 = _pair_split(e_lo, e_hi)
  ep_ref[0] = p0
  ep_ref[1] = p1
  ew_lo = jnp.dot(e_lo, wc_ref[...], preferred_element_type=F32) + be_ref[...]
  ew_hi = jnp.dot(e_hi, wc_ref[...], preferred_element_type=F32) + be_ref[...]
  q0, q1 = _pair_split(ew_lo, ew_hi)
  ewp_ref[0] = q0
  ewp_ref[1] = q1


def _dec_reduce_body(pacc_ref, bd2_ref, p_ref):
  p_ref[...] = jnp.sum(pacc_ref[...], axis=1, keepdims=True) + bd2_ref[...]


def _sc_layer_body(src, dst, tav, tbsw, ep, ewp, eop, aggsig,
                   slv, shv, dlv, dhv, ialo, iahi, iblo, ibhi,
                   avlo, avhi, blo, bhi, epb, ewpb,
                   shared, sem1, sem2, sem3, sem4):
  c = lax.axis_index("c")
  s = lax.axis_index("s")
  cn = c * N

  def zrow(r, _):
    for k2 in range(8):
      epb[r, pl.ds(k2 * 16, 16)] = jnp.zeros((16,), F32)
    return 0
  lax.fori_loop(0, EB, zrow, 0)
  nbase = s * NZ
  for z in range(0, NZ, EB):
    rows = min(EB, NZ - z)
    pltpu.sync_copy(epb.at[pl.ds(0, rows)], shared.at[pl.ds(nbase + z, rows)])

  @pl.when(s == NSUB - 1)
  def _():
    pltpu.sync_copy(epb.at[pl.ds(0, N - NZ * NSUB)],
                    shared.at[pl.ds(NZ * NSUB, N - NZ * NSUB)])
  plsc.subcore_barrier()

  def do_block(j):
    base = j * EB
    d1 = pltpu.async_copy(src.at[pl.ds(base, EB)], slv, sem1)
    d2 = pltpu.async_copy(src.at[pl.ds(EH + base, EB)], shv, sem1)
    d3 = pltpu.async_copy(dst.at[pl.ds(base, EB)], dlv, sem1)
    d4 = pltpu.async_copy(dst.at[pl.ds(EH + base, EB)], dhv, sem1)
    d5 = pltpu.async_copy(ep.at[c, pl.ds(base, EB)], epb, sem2)
    d6 = pltpu.async_copy(ewp.at[c, pl.ds(base, EB)], ewpb, sem2)
    d1.wait()
    d2.wait()
    d3.wait()
    d4.wait()
    for k2 in range(EB // 16):
      sl = pl.ds(k2 * 16, 16)
      ialo[sl] = slv[sl] + cn
      iahi[sl] = shv[sl] + cn
      iblo[sl] = dlv[sl] + cn
      ibhi[sl] = dhv[sl] + cn
    g1 = pltpu.async_copy(tav.at[ialo], avlo, sem3)
    g2 = pltpu.async_copy(tav.at[iahi], avhi, sem3)
    g3 = pltpu.async_copy(tbsw.at[iblo], blo, sem4)
    g4 = pltpu.async_copy(tbsw.at[ibhi], bhi, sem4)
    d5.wait()
    d6.wait()
    g1.wait()
    g2.wait()
    g3.wait()
    g4.wait()

    def pair(r, _):
      for k2 in range(H // 16):
        sl = pl.ds(k2 * 16, 16)
        sv = pl.ds(H + k2 * 16, 16)
        en = avlo[r, sl] + blo[r, sl] + ewpb[r, sl]
        sg = 1.0 / (1.0 + jnp.exp(-en))
        avlo[r, sl] = sg * avlo[r, sv]
        avlo[r, sv] = sg
        epb[r, sl] = epb[r, sl] + jnp.maximum(en, 0.0)
        en2 = avhi[r, sl] + bhi[r, sl] + ewpb[r, sv]
        sg2 = 1.0 / (1.0 + jnp.exp(-en2))
        avhi[r, sl] = sg2 * avhi[r, sv]
        avhi[r, sv] = sg2
        epb[r, sv] = epb[r, sv] + jnp.maximum(en2, 0.0)
      return 0
    lax.fori_loop(0, EB, pair, 0)

    pltpu.sync_copy(epb, eop.at[c, pl.ds(base, EB)])
    pltpu.sync_copy(avlo, shared.at[dlv], add=True)
    pltpu.sync_copy(avhi, shared.at[dhv], add=True)

  def block(i, _):
    do_block(s + i * NSUB)
    return 0
  nfull = NBLK_L // NSUB
  lax.fori_loop(0, nfull, block, 0)
  nrem = NBLK_L - nfull * NSUB

  @pl.when(s < nrem)
  def _():
    do_block(nfull * NSUB + s)
  plsc.subcore_barrier()
  pltpu.sync_copy(shared.at[pl.ds(nbase, NZ)], aggsig.at[c, pl.ds(nbase, NZ)])

  @pl.when(s == NSUB - 1)
  def _():
    pltpu.sync_copy(shared.at[pl.ds(NZ * NSUB, N - NZ * NSUB)],
                    aggsig.at[c, pl.ds(NZ * NSUB, N - NZ * NSUB)])


def _sc_dec_body(src, dst, da, db, qp, wd2, pacc,
                 slv, shv, dlv, dhv, alo, ahi, blo, bhi, q0b, q1b,
                 wv, pblo, pbhi, sem1, sem2, sem3, sem4):
  c = lax.axis_index("c")
  s = lax.axis_index("s")
  w = s * NCORE + c
  pltpu.sync_copy(wd2, wv)

  def do_block(jb):
    base = jb * EBD
    d1 = pltpu.async_copy(src.at[pl.ds(base, EBD)], slv, sem1)
    d2 = pltpu.async_copy(src.at[pl.ds(EH + base, EBD)], shv, sem1)
    d3 = pltpu.async_copy(dst.at[pl.ds(base, EBD)], dlv, sem1)
    d4 = pltpu.async_copy(dst.at[pl.ds(EH + base, EBD)], dhv, sem1)
    d5 = pltpu.async_copy(qp.at[0, pl.ds(base, EBD)], q0b, sem2)
    d6 = pltpu.async_copy(qp.at[1, pl.ds(base, EBD)], q1b, sem2)
    d1.wait()
    d2.wait()
    d3.wait()
    d4.wait()
    g1 = pltpu.async_copy(da.at[slv], alo, sem3)
    g2 = pltpu.async_copy(da.at[shv], ahi, sem3)
    g3 = pltpu.async_copy(db.at[dlv], blo, sem4)
    g4 = pltpu.async_copy(db.at[dhv], bhi, sem4)
    d5.wait()
    d6.wait()
    g1.wait()
    g2.wait()
    g3.wait()
    g4.wait()

    def pair(r, _):
      acc_lo = jnp.zeros((16,), F32)
      acc_hi = jnp.zeros((16,), F32)
      for k2 in range(D // 16):
        sl = pl.ds(k2 * 16, 16)
        if k2 < 4:
          q_lo = q0b[r, pl.ds(k2 * 16, 16)]
          q_hi = q0b[r, pl.ds(H + k2 * 16, 16)]
        else:
          q_lo = q1b[r, pl.ds((k2 - 4) * 16, 16)]
          q_hi = q1b[r, pl.ds(H + (k2 - 4) * 16, 16)]
        wk = wv[sl]
        acc_lo = acc_lo + wk * jnp.maximum(q_lo + alo[r, sl] + blo[r, sl], 0.0)
        acc_hi = acc_hi + wk * jnp.maximum(q_hi + ahi[r, sl] + bhi[r, sl], 0.0)
      pblo[r, :] = acc_lo
      pbhi[r, :] = acc_hi
      return 0
    lax.fori_loop(0, EBD, pair, 0)
    pltpu.sync_copy(pblo, pacc.at[pl.ds(base, EBD)])
    pltpu.sync_copy(pbhi, pacc.at[pl.ds(EH + base, EBD)])

  def block(i, _):
    do_block(w + i * (NCORE * NSUB))
    return 0
  nfull = NBLK_ALL // (NCORE * NSUB)
  lax.fori_loop(0, nfull, block, 0)
  nrem = NBLK_ALL - nfull * (NCORE * NSUB)

  @pl.when(w < nrem)
  def _():
    do_block(nfull * (NCORE * NSUB) + w)


def _mk_sc_layer():
  mesh = plsc.VectorSubcoreMesh(core_axis_name="c", subcore_axis_name="s")
  return pl.kernel(
      _sc_layer_body,
      out_type=[jax.ShapeDtypeStruct((2, EH, D), F32),
                jax.ShapeDtypeStruct((2, N, D), F32)],
      mesh=mesh,
      scratch_types=[
          pltpu.VMEM((EB,), jnp.int32),
          pltpu.VMEM((EB,), jnp.int32),
          pltpu.VMEM((EB,), jnp.int32),
          pltpu.VMEM((EB,), jnp.int32),
          pltpu.VMEM((EB,), jnp.int32),
          pltpu.VMEM((EB,), jnp.int32),
          pltpu.VMEM((EB,), jnp.int32),
          pltpu.VMEM((EB,), jnp.int32),
          pltpu.VMEM((EB, D), F32),
          pltpu.VMEM((EB, D), F32),
          pltpu.VMEM((EB, D), F32),
          pltpu.VMEM((EB, D), F32),
          pltpu.VMEM((EB, D), F32),
          pltpu.VMEM((EB, D), F32),
          pltpu.VMEM_SHARED((N, D), F32),
          pltpu.SemaphoreType.DMA,
          pltpu.SemaphoreType.DMA,
          pltpu.SemaphoreType.DMA,
          pltpu.SemaphoreType.DMA,
      ],
  )


def _mk_sc_dec():
  mesh = plsc.VectorSubcoreMesh(core_axis_name="c", subcore_axis_name="s")
  return pl.kernel(
      _sc_dec_body,
      out_type=jax.ShapeDtypeStruct((E, 16), F32),
      mesh=mesh,
      scratch_types=[
          pltpu.VMEM((EBD,), jnp.int32),
          pltpu.VMEM((EBD,), jnp.int32),
          pltpu.VMEM((EBD,), jnp.int32),
          pltpu.VMEM((EBD,), jnp.int32),
          pltpu.VMEM((EBD, D), F32),
          pltpu.VMEM((EBD, D), F32),
          pltpu.VMEM((EBD, D), F32),
          pltpu.VMEM((EBD, D), F32),
          pltpu.VMEM((EBD, D), F32),
          pltpu.VMEM((EBD, D), F32),
          pltpu.VMEM((D,), F32),
          pltpu.VMEM((EBD, 16), F32),
          pltpu.VMEM((EBD, 16), F32),
          pltpu.SemaphoreType.DMA,
          pltpu.SemaphoreType.DMA,
          pltpu.SemaphoreType.DMA,
          pltpu.SemaphoreType.DMA,
      ],
  )


def kernel(reads, edge_index, overlap_similarity, overlap_length, Wc, bc, We,
           be, WA, WB, WC, bE, WU, WV, bH, Wd1, bd1, Wd2, bd2):
  BN = 1000
  BEP = 400
  grid_n = N // BN
  grid_ep = EH // BEP

  wck = jnp.transpose(Wc, (1, 2, 0))
  def mk_wav(l):
    return jnp.stack([
        jnp.concatenate([WA[l][:, :H], WV[l][:, :H]], axis=1),
        jnp.concatenate([WA[l][:, H:], WV[l][:, H:]], axis=1)])
  wav_dec = jnp.stack([Wd1[:D], Wd1[D:2 * D]])

  src = edge_index[0]
  dst = edge_index[1]
  sim = overlap_similarity.reshape(E, 1)
  ln = overlap_length.reshape(E, 1)

  row = lambda *shape: pl.BlockSpec(
      shape, lambda i: tuple([i] + [0] * (len(shape) - 1)))
  row2 = lambda *shape: pl.BlockSpec(
      shape, lambda i: tuple([0, i] + [0] * (len(shape) - 2)))
  whole = lambda *shape: pl.BlockSpec(shape, lambda i: (0,) * len(shape))
  hi_blocks = grid_ep

  enc = pl.pallas_call(
      _encoder_body,
      grid=(grid_n,),
      in_specs=[row(BN, 4, READ_LEN), whole(4, K, D), whole(D),
                whole(2, D, D), whole(D, D), whole(D, D), whole(D)],
      out_specs=[row(BN, D), row2(2, BN, D), row2(2, BN, D), row(BN, D)],
      out_shape=[jax.ShapeDtypeStruct((N, D), F32),
                 jax.ShapeDtypeStruct((2, N, D), F32),
                 jax.ShapeDtypeStruct((2, N, D), F32),
                 jax.ShapeDtypeStruct((N, D), F32)],
  )
  h, tav, tbsw, hu = enc(reads, wck, bc, mk_wav(0), WB[0], WU[0], bH[0])

  lo_spec = pl.BlockSpec((BEP, 1), lambda i: (i, 0))
  hi_spec = pl.BlockSpec((BEP, 1), lambda i: (i + hi_blocks, 0))
  edge0 = pl.pallas_call(
      _edge0_body,
      grid=(grid_ep,),
      in_specs=[lo_spec, hi_spec, lo_spec, hi_spec, whole(2, D), whole(D),
                whole(D, D), whole(D)],
      out_specs=[row2(2, BEP, D), row2(2, BEP, D)],
      out_shape=[jax.ShapeDtypeStruct((2, EH, D), F32),
                 jax.ShapeDtypeStruct((2, EH, D), F32)],
  )
  ep, ewp = edge0(sim, sim, ln, ln, We, be, WC[0], bE[0])

  edge_mm = pl.pallas_call(
      _edge_mm_body,
      grid=(grid_ep,),
      in_specs=[row2(2, BEP, D), whole(D, D), whole(D)],
      out_specs=row2(2, BEP, D),
      out_shape=jax.ShapeDtypeStruct((2, EH, D), F32),
  )

  node = pl.pallas_call(
      _node_body,
      grid=(grid_n,),
      in_specs=[row(BN, D), row(BN, D), row2(2, BN, D),
                whole(2, D, D), whole(D, D), whole(D, D), whole(D)],
      out_specs=[row(BN, D), row2(2, BN, D), row2(2, BN, D), row(BN, D)],
      out_shape=[jax.ShapeDtypeStruct((N, D), F32),
                 jax.ShapeDtypeStruct((2, N, D), F32),
                 jax.ShapeDtypeStruct((2, N, D), F32),
                 jax.ShapeDtypeStruct((N, D), F32)],
  )

  sc_layer = _mk_sc_layer()
  for l in range(NUM_LAYERS):
    tav2 = tav.reshape(2 * N, D)
    tbsw2 = tbsw.reshape(2 * N, D)
    ep, aggsig = sc_layer(src, dst, tav2, tbsw2, ep, ewp)
    if l < NUM_LAYERS - 1:
      h, tav, tbsw, hu = node(h, hu, aggsig, mk_wav(l + 1), WB[l + 1],
                              WU[l + 1], bH[l + 1])
      ewp = edge_mm(ep, WC[l + 1], bE[l + 1])
    else:
      zD = jnp.zeros((D, D), F32)
      _, dadb, _, _ = node(h, hu, aggsig, wav_dec, zD, zD, jnp.zeros((D,), F32))
      qp = edge_mm(ep, Wd1[2 * D:], bd1)

  sc_dec = _mk_sc_dec()
  wd2f = Wd2.reshape(D)
  pacc = sc_dec(src, dst, dadb[0], dadb[1], qp, wd2f)

  dec_reduce = pl.pallas_call(
      _dec_reduce_body,
      grid=(E // 1000,),
      in_specs=[row(1000, 16), whole(1)],
      out_specs=row(1000, 1),
      out_shape=jax.ShapeDtypeStruct((E, 1), F32),
  )
  return dec_reduce(pacc, bd2)


# speedup vs baseline: 2.7081x; 1.1476x over previous
import jax
import jax.numpy as jnp
from jax import lax
from jax.experimental import pallas as pl
from jax.experimental.pallas import tpu as pltpu
from jax.experimental.pallas import tpu_sc as plsc

N = 10000
E = 320000
D = 128
K = 16
READ_LEN = 128
NUM_LAYERS = 4
H = D // 2
T = (READ_LEN - K) // 4 + 1

NSUB = 16
NCORE = 2
EH = E // 2
EB = 32
EBD = 80
NBLK_L = EH // EB
NBLK_ALL = EH // EBD
NZ = 624

F32 = jnp.float32
HIGH = lax.Precision.HIGHEST


def _proj(h, wav_ref, wb_ref, wu_ref, bh_ref):
  tav0 = jnp.dot(h, wav_ref[0], preferred_element_type=F32)
  tav1 = jnp.dot(h, wav_ref[1], preferred_element_type=F32)
  hb = jnp.dot(h, wb_ref[...], preferred_element_type=F32)
  hbs = jnp.concatenate([hb[:, H:], hb[:, :H]], axis=1)
  hu = jnp.dot(h, wu_ref[...], preferred_element_type=F32)
  hu = hu + bh_ref[...]
  return tav0, tav1, hb, hbs, hu


def _encoder_body(reads_ref, wck_ref, bc_ref, wav_ref, wb_ref, wu_ref, bh_ref,
                  h_ref, tav_ref, tb_ref, hu_ref):
  bn = reads_ref.shape[0]
  acc = jnp.zeros((bn, D), F32)
  bc = bc_ref[...]
  w2 = jnp.concatenate([wck_ref[c] for c in range(4)], axis=0)
  for t in range(T):
    win = jnp.concatenate([reads_ref[:, c, 4 * t:4 * t + K] for c in range(4)],
                          axis=1)
    s = jnp.dot(win, w2, preferred_element_type=F32)
    acc = acc + jnp.maximum(s + bc, 0.0)
  h = acc * (1.0 / T)
  tav0, tav1, hb, hbs, hu = _proj(h, wav_ref, wb_ref, wu_ref, bh_ref)
  h_ref[...] = h
  tav_ref[0] = tav0
  tav_ref[1] = tav1
  tb_ref[0] = hb
  tb_ref[1] = hbs
  hu_ref[...] = hu


def _node_body(h_ref, hu_ref, ag_ref, wav_ref, wb_ref, wu_ref, bh_ref,
               h2_ref, tav_ref, tb_ref, hu2_ref):
  agg = jnp.concatenate([ag_ref[0, :, :H], ag_ref[1, :, :H]], axis=1)
  nrm = jnp.concatenate([ag_ref[0, :, H:], ag_ref[1, :, H:]], axis=1)
  h = h_ref[...] + jnp.maximum(hu_ref[...] + agg / (nrm + 1e-6), 0.0)
  tav0, tav1, hb, hbs, hu = _proj(h, wav_ref, wb_ref, wu_ref, bh_ref)
  h2_ref[...] = h
  tav_ref[0] = tav0
  tav_ref[1] = tav1
  tb_ref[0] = hb
  tb_ref[1] = hbs
  hu2_ref[...] = hu


def _pair_split(lo, hi):
  p0 = jnp.concatenate([lo[:, :H], hi[:, :H]], axis=1)
  p1 = jnp.concatenate([lo[:, H:], hi[:, H:]], axis=1)
  return p0, p1


def _pair_join(p0, p1):
  lo = jnp.concatenate([p0[:, :H], p1[:, :H]], axis=1)
  hi = jnp.concatenate([p0[:, H:], p1[:, H:]], axis=1)
  return lo, hi


def _edge_mm_body(ep_ref, wc_ref, be_ref, ewp_ref):
  lo, hi = _pair_join(ep_ref[0], ep_ref[1])
  ew_lo = jnp.dot(lo, wc_ref[...], preferred_element_type=F32) + be_ref[...]
  ew_hi = jnp.dot(hi, wc_ref[...], preferred_element_type=F32) + be_ref[...]
  p0, p1 = _pair_split(ew_lo, ew_hi)
  ewp_ref[0] = p0
  ewp_ref[1] = p1


def _edge0_body(sim_lo, sim_hi, ln_lo, ln_hi, we_ref, be0_ref, wc_ref, be_ref,
                ep_ref, ewp_ref):
  def enc(sim, ln):
    return jnp.maximum(sim[...] * we_ref[0:1, :] + ln[...] * we_ref[1:2, :]
                       + be0_ref[...], 0.0)
  e_lo = enc(sim_lo, ln_lo)
  e_hi = enc(sim_hi, ln_hi)
  p0, p1 = _pair_split(e_lo, e_hi)
  ep_ref[0] = p0
  ep_ref[1] = p1
  ew_lo = jnp.dot(e_lo, wc_ref[...], preferred_element_type=F32) + be_ref[...]
  ew_hi = jnp.dot(e_hi, wc_ref[...], preferred_element_type=F32) + be_ref[...]
  q0, q1 = _pair_split(ew_lo, ew_hi)
  ewp_ref[0] = q0
  ewp_ref[1] = q1


def _dec_reduce_body(pacc_ref, bd2_ref, p_ref):
  p_ref[...] = jnp.sum(pacc_ref[...], axis=1, keepdims=True) + bd2_ref[...]


def _sc_layer_body(src, dst, tav, tbsw, ep, ewp, eop, aggsig, *scr):
  shared = scr[-1]
  sets = (scr[0:15], scr[15:30])
  c = lax.axis_index("c")
  s = lax.axis_index("s")
  cn = c * N

  epb0 = sets[0][10]
  def zrow(r, _):
    for k2 in range(8):
      epb0[r, pl.ds(k2 * 16, 16)] = jnp.zeros((16,), F32)
    return 0
  lax.fori_loop(0, EB, zrow, 0)
  nbase = s * NZ
  for z in range(0, NZ, EB):
    rows = min(EB, NZ - z)
    pltpu.sync_copy(epb0.at[pl.ds(0, rows)], shared.at[pl.ds(nbase + z, rows)])

  @pl.when(s == NSUB - 1)
  def _():
    pltpu.sync_copy(epb0.at[pl.ds(0, N - NZ * NSUB)],
                    shared.at[pl.ds(NZ * NSUB, N - NZ * NSUB)])
  plsc.subcore_barrier()

  LAST = NBLK_L - 1

  def issue_idx(j, S):
    base = j * EB
    pltpu.async_copy(src.at[pl.ds(base, EB)], S[0], S[12])
    pltpu.async_copy(src.at[pl.ds(EH + base, EB)], S[1], S[12])
    pltpu.async_copy(dst.at[pl.ds(base, EB)], S[2], S[12])
    pltpu.async_copy(dst.at[pl.ds(EH + base, EB)], S[3], S[12])

  def wait_idx(S):
    for b in (S[0], S[1], S[2], S[3]):
      pltpu.make_async_copy(src.at[pl.ds(0, EB)], b, S[12]).wait()

  def arith(S):
    for k2 in range(EB // 16):
      sl = pl.ds(k2 * 16, 16)
      S[0][sl] = S[0][sl] + cn
      S[1][sl] = S[1][sl] + cn
      S[4][sl] = S[2][sl] + cn
      S[5][sl] = S[3][sl] + cn

  def issue_ew(j, S):
    base = j * EB
    pltpu.async_copy(ep.at[c, pl.ds(base, EB)], S[10], S[13])
    pltpu.async_copy(ewp.at[c, pl.ds(base, EB)], S[11], S[13])

  def wait_ew(S):
    for b in (S[10], S[11]):
      pltpu.make_async_copy(ep.at[c, pl.ds(0, EB)], b, S[13]).wait()

  def issue_gather(S):
    pltpu.async_copy(tav.at[S[0]], S[6], S[14])
    pltpu.async_copy(tav.at[S[1]], S[7], S[14])
    pltpu.async_copy(tbsw.at[S[4]], S[8], S[14])
    pltpu.async_copy(tbsw.at[S[5]], S[9], S[14])

  def wait_gather(S):
    for b in (S[6], S[7], S[8], S[9]):
      pltpu.make_async_copy(tav.at[S[0]], b, S[14]).wait()

  def compute(S):
    avlo, avhi, blo, bhi, epb, ewpb = S[6], S[7], S[8], S[9], S[10], S[11]
    def pair(r, _):
      for k2 in range(H // 16):
        sl = pl.ds(k2 * 16, 16)
        sv = pl.ds(H + k2 * 16, 16)
        en = avlo[r, sl] + blo[r, sl] + ewpb[r, sl]
        sg = 1.0 / (1.0 + jnp.exp(-en))
        avlo[r, sl] = sg * avlo[r, sv]
        avlo[r, sv] = sg
        epb[r, sl] = epb[r, sl] + jnp.maximum(en, 0.0)
        en2 = avhi[r, sl] + bhi[r, sl] + ewpb[r, sv]
        sg2 = 1.0 / (1.0 + jnp.exp(-en2))
        avhi[r, sl] = sg2 * avhi[r, sv]
        avhi[r, sv] = sg2
        epb[r, sv] = epb[r, sv] + jnp.maximum(en2, 0.0)
      return 0
    lax.fori_loop(0, EB, pair, 0)

  def writes(j, S):
    base = j * EB
    pltpu.sync_copy(S[10], eop.at[c, pl.ds(base, EB)])
    pltpu.sync_copy(S[6], shared.at[S[2]], add=True)
    pltpu.sync_copy(S[7], shared.at[S[3]], add=True)

  def jid(i):
    return jnp.minimum(s + i * NSUB, LAST)

  nfull = NBLK_L // NSUB

  issue_idx(jid(0), sets[0])
  issue_ew(jid(0), sets[0])
  wait_idx(sets[0])
  arith(sets[0])
  issue_gather(sets[0])
  issue_idx(jid(1), sets[1])

  def sub(i, CUR, NXT):
    wait_idx(NXT)
    arith(NXT)
    issue_ew(jid(i + 1), NXT)
    issue_gather(NXT)
    wait_gather(CUR)
    wait_ew(CUR)
    compute(CUR)
    writes(jid(i), CUR)
    issue_idx(jid(i + 2), CUR)

  def body(ii, _):
    i = ii * 2
    sub(i, sets[0], sets[1])
    sub(i + 1, sets[1], sets[0])
    return 0
  lax.fori_loop(0, nfull // 2, body, 0)

  wait_gather(sets[0])
  wait_ew(sets[0])
  wait_idx(sets[1])

  nrem = NBLK_L - nfull * NSUB
  @pl.when(s < nrem)
  def _():
    S = sets[0]
    j = nfull * NSUB + s
    issue_idx(j, S)
    issue_ew(j, S)
    wait_idx(S)
    arith(S)
    issue_gather(S)
    wait_gather(S)
    wait_ew(S)
    compute(S)
    writes(j, S)

  plsc.subcore_barrier()
  pltpu.sync_copy(shared.at[pl.ds(nbase, NZ)], aggsig.at[c, pl.ds(nbase, NZ)])

  @pl.when(s == NSUB - 1)
  def _():
    pltpu.sync_copy(shared.at[pl.ds(NZ * NSUB, N - NZ * NSUB)],
                    aggsig.at[c, pl.ds(NZ * NSUB, N - NZ * NSUB)])


def _sc_dec_body(src, dst, da, db, qp, wd2, pacc,
                 slv, shv, dlv, dhv, alo, ahi, blo, bhi, q0b, q1b,
                 wv, pblo, pbhi, sem1, sem2, sem3, sem4):
  c = lax.axis_index("c")
  s = lax.axis_index("s")
  w = s * NCORE + c
  pltpu.sync_copy(wd2, wv)

  def do_block(jb):
    base = jb * EBD
    d1 = pltpu.async_copy(src.at[pl.ds(base, EBD)], slv, sem1)
    d2 = pltpu.async_copy(src.at[pl.ds(EH + base, EBD)], shv, sem1)
    d3 = pltpu.async_copy(dst.at[pl.ds(base, EBD)], dlv, sem1)
    d4 = pltpu.async_copy(dst.at[pl.ds(EH + base, EBD)], dhv, sem1)
    d5 = pltpu.async_copy(qp.at[0, pl.ds(base, EBD)], q0b, sem2)
    d6 = pltpu.async_copy(qp.at[1, pl.ds(base, EBD)], q1b, sem2)
    d1.wait()
    d2.wait()
    d3.wait()
    d4.wait()
    g1 = pltpu.async_copy(da.at[slv], alo, sem3)
    g2 = pltpu.async_copy(da.at[shv], ahi, sem3)
    g3 = pltpu.async_copy(db.at[dlv], blo, sem4)
    g4 = pltpu.async_copy(db.at[dhv], bhi, sem4)
    d5.wait()
    d6.wait()
    g1.wait()
    g2.wait()
    g3.wait()
    g4.wait()

    def pair(r, _):
      acc_lo = jnp.zeros((16,), F32)
      acc_hi = jnp.zeros((16,), F32)
      for k2 in range(D // 16):
        sl = pl.ds(k2 * 16, 16)
        if k2 < 4:
          q_lo = q0b[r, pl.ds(k2 * 16, 16)]
          q_hi = q0b[r, pl.ds(H + k2 * 16, 16)]
        else:
          q_lo = q1b[r, pl.ds((k2 - 4) * 16, 16)]
          q_hi = q1b[r, pl.ds(H + (k2 - 4) * 16, 16)]
        wk = wv[sl]
        acc_lo = acc_lo + wk * jnp.maximum(q_lo + alo[r, sl] + blo[r, sl], 0.0)
        acc_hi = acc_hi + wk * jnp.maximum(q_hi + ahi[r, sl] + bhi[r, sl], 0.0)
      pblo[r, :] = acc_lo
      pbhi[r, :] = acc_hi
      return 0
    lax.fori_loop(0, EBD, pair, 0)
    pltpu.sync_copy(pblo, pacc.at[pl.ds(base, EBD)])
    pltpu.sync_copy(pbhi, pacc.at[pl.ds(EH + base, EBD)])

  def block(i, _):
    do_block(w + i * (NCORE * NSUB))
    return 0
  nfull = NBLK_ALL // (NCORE * NSUB)
  lax.fori_loop(0, nfull, block, 0)
  nrem = NBLK_ALL - nfull * (NCORE * NSUB)

  @pl.when(w < nrem)
  def _():
    do_block(nfull * (NCORE * NSUB) + w)


def _mk_sc_layer():
  mesh = plsc.VectorSubcoreMesh(core_axis_name="c", subcore_axis_name="s")
  return pl.kernel(
      _sc_layer_body,
      out_type=[jax.ShapeDtypeStruct((2, EH, D), F32),
                jax.ShapeDtypeStruct((2, N, D), F32)],
      mesh=mesh,
      scratch_types=(
          [pltpu.VMEM((EB,), jnp.int32) for _ in range(6)]
          + [pltpu.VMEM((EB, D), F32) for _ in range(6)]
          + [pltpu.SemaphoreType.DMA] * 3
          + [pltpu.VMEM((EB,), jnp.int32) for _ in range(6)]
          + [pltpu.VMEM((EB, D), F32) for _ in range(6)]
          + [pltpu.SemaphoreType.DMA] * 3
          + [pltpu.VMEM_SHARED((N, D), F32)]
      ),
  )


def _mk_sc_dec():
  mesh = plsc.VectorSubcoreMesh(core_axis_name="c", subcore_axis_name="s")
  return pl.kernel(
      _sc_dec_body,
      out_type=jax.ShapeDtypeStruct((E, 16), F32),
      mesh=mesh,
      scratch_types=[
          pltpu.VMEM((EBD,), jnp.int32),
          pltpu.VMEM((EBD,), jnp.int32),
          pltpu.VMEM((EBD,), jnp.int32),
          pltpu.VMEM((EBD,), jnp.int32),
          pltpu.VMEM((EBD, D), F32),
          pltpu.VMEM((EBD, D), F32),
          pltpu.VMEM((EBD, D), F32),
          pltpu.VMEM((EBD, D), F32),
          pltpu.VMEM((EBD, D), F32),
          pltpu.VMEM((EBD, D), F32),
          pltpu.VMEM((D,), F32),
          pltpu.VMEM((EBD, 16), F32),
          pltpu.VMEM((EBD, 16), F32),
          pltpu.SemaphoreType.DMA,
          pltpu.SemaphoreType.DMA,
          pltpu.SemaphoreType.DMA,
          pltpu.SemaphoreType.DMA,
      ],
  )


def kernel(reads, edge_index, overlap_similarity, overlap_length, Wc, bc, We,
           be, WA, WB, WC, bE, WU, WV, bH, Wd1, bd1, Wd2, bd2):
  BN = 1000
  BEP = 400
  grid_n = N // BN
  grid_ep = EH // BEP

  wck = jnp.transpose(Wc, (1, 2, 0))
  def mk_wav(l):
    return jnp.stack([
        jnp.concatenate([WA[l][:, :H], WV[l][:, :H]], axis=1),
        jnp.concatenate([WA[l][:, H:], WV[l][:, H:]], axis=1)])
  wav_dec = jnp.stack([Wd1[:D], Wd1[D:2 * D]])

  src = edge_index[0]
  dst = edge_index[1]
  sim = overlap_similarity.reshape(E, 1)
  ln = overlap_length.reshape(E, 1)

  row = lambda *shape: pl.BlockSpec(
      shape, lambda i: tuple([i] + [0] * (len(shape) - 1)))
  row2 = lambda *shape: pl.BlockSpec(
      shape, lambda i: tuple([0, i] + [0] * (len(shape) - 2)))
  whole = lambda *shape: pl.BlockSpec(shape, lambda i: (0,) * len(shape))
  hi_blocks = grid_ep

  enc = pl.pallas_call(
      _encoder_body,
      grid=(grid_n,),
      in_specs=[row(BN, 4, READ_LEN), whole(4, K, D), whole(D),
                whole(2, D, D), whole(D, D), whole(D, D), whole(D)],
      out_specs=[row(BN, D), row2(2, BN, D), row2(2, BN, D), row(BN, D)],
      out_shape=[jax.ShapeDtypeStruct((N, D), F32),
                 jax.ShapeDtypeStruct((2, N, D), F32),
                 jax.ShapeDtypeStruct((2, N, D), F32),
                 jax.ShapeDtypeStruct((N, D), F32)],
  )
  h, tav, tbsw, hu = enc(reads, wck, bc, mk_wav(0), WB[0], WU[0], bH[0])

  lo_spec = pl.BlockSpec((BEP, 1), lambda i: (i, 0))
  hi_spec = pl.BlockSpec((BEP, 1), lambda i: (i + hi_blocks, 0))
  edge0 = pl.pallas_call(
      _edge0_body,
      grid=(grid_ep,),
      in_specs=[lo_spec, hi_spec, lo_spec, hi_spec, whole(2, D), whole(D),
                whole(D, D), whole(D)],
      out_specs=[row2(2, BEP, D), row2(2, BEP, D)],
      out_shape=[jax.ShapeDtypeStruct((2, EH, D), F32),
                 jax.ShapeDtypeStruct((2, EH, D), F32)],
  )
  ep, ewp = edge0(sim, sim, ln, ln, We, be, WC[0], bE[0])

  edge_mm = pl.pallas_call(
      _edge_mm_body,
      grid=(grid_ep,),
      in_specs=[row2(2, BEP, D), whole(D, D), whole(D)],
      out_specs=row2(2, BEP, D),
      out_shape=jax.ShapeDtypeStruct((2, EH, D), F32),
  )

  node = pl.pallas_call(
      _node_body,
      grid=(grid_n,),
      in_specs=[row(BN, D), row(BN, D), row2(2, BN, D),
                whole(2, D, D), whole(D, D), whole(D, D), whole(D)],
      out_specs=[row(BN, D), row2(2, BN, D), row2(2, BN, D), row(BN, D)],
      out_shape=[jax.ShapeDtypeStruct((N, D), F32),
                 jax.ShapeDtypeStruct((2, N, D), F32),
                 jax.ShapeDtypeStruct((2, N, D), F32),
                 jax.ShapeDtypeStruct((N, D), F32)],
  )

  sc_layer = _mk_sc_layer()
  for l in range(NUM_LAYERS):
    tav2 = tav.reshape(2 * N, D)
    tbsw2 = tbsw.reshape(2 * N, D)
    ep, aggsig = sc_layer(src, dst, tav2, tbsw2, ep, ewp)
    if l < NUM_LAYERS - 1:
      h, tav, tbsw, hu = node(h, hu, aggsig, mk_wav(l + 1), WB[l + 1],
                              WU[l + 1], bH[l + 1])
      ewp = edge_mm(ep, WC[l + 1], bE[l + 1])
    else:
      zD = jnp.zeros((D, D), F32)
      _, dadb, _, _ = node(h, hu, aggsig, wav_dec, zD, zD, jnp.zeros((D,), F32))
      qp = edge_mm(ep, Wd1[2 * D:], bd1)

  sc_dec = _mk_sc_dec()
  wd2f = Wd2.reshape(D)
  pacc = sc_dec(src, dst, dadb[0], dadb[1], qp, wd2f)

  dec_reduce = pl.pallas_call(
      _dec_reduce_body,
      grid=(E // 1000,),
      in_specs=[row(1000, 16), whole(1)],
      out_specs=row(1000, 1),
      out_shape=jax.ShapeDtypeStruct((E, 1), F32),
  )
  return dec_reduce(pacc, bd2)
